# Initial kernel scaffold; baseline (speedup 1.0000x reference)
import functools

import jax
import jax.numpy as jnp
from jax import lax
from jax.experimental import pallas as pl
from jax.experimental.pallas import tpu as pltpu
from jax.experimental.pallas import tpu_sc as plsc

_NC = 2
_NS = 16
_NW = _NC * _NS
_CH = 128


def _make_deg(e_pad, n_pad):
  e_per_w = e_pad // _NW
  n_chunks = e_per_w // _CH
  rpt = n_pad // _NS
  mesh = plsc.VectorSubcoreMesh(core_axis_name="c", subcore_axis_name="s")

  @functools.partial(
      pl.kernel,
      mesh=mesh,
      out_type=jax.ShapeDtypeStruct((_NC, n_pad), jnp.float32),
      scratch_types=[
          pltpu.VMEM((_CH,), jnp.int32),
          pltpu.VMEM((_CH,), jnp.float32),
          pltpu.VMEM_SHARED((n_pad,), jnp.float32),
          pltpu.SemaphoreType.DMA,
      ],
  )
  def deg_kernel(dst_hbm, ones_hbm, zeros_hbm, out_hbm, idx_v, ones_v,
                 acc_sh, sem):
    c = lax.axis_index("c")
    s = lax.axis_index("s")
    wid = c * _NS + s
    pltpu.sync_copy(zeros_hbm.at[pl.ds(s * rpt, rpt)],
                    acc_sh.at[pl.ds(s * rpt, rpt)])
    pltpu.sync_copy(ones_hbm, ones_v)
    plsc.subcore_barrier()

    def step(i, carry):
      base = wid * e_per_w + i * _CH
      pltpu.sync_copy(dst_hbm.at[pl.ds(base, _CH)], idx_v)
      pltpu.sync_copy(ones_v, acc_sh.at[idx_v], add=True)
      return carry

    lax.fori_loop(0, n_chunks, step, 0)
    plsc.subcore_barrier()
    pltpu.sync_copy(acc_sh.at[pl.ds(s * rpt, rpt)],
                    out_hbm.at[c, pl.ds(s * rpt, rpt)])

  return deg_kernel


def _make_agg(e_pad, n_pad, d):
  e_per_w = e_pad // _NW
  n_chunks = e_per_w // _CH
  rpt = n_pad // _NS
  mesh = plsc.VectorSubcoreMesh(core_axis_name="c", subcore_axis_name="s")

  @functools.partial(
      pl.kernel,
      mesh=mesh,
      out_type=jax.ShapeDtypeStruct((_NC, n_pad, d), jnp.float32),
      scratch_types=[
          pltpu.VMEM((_CH,), jnp.int32),
          pltpu.VMEM((_CH,), jnp.int32),
          pltpu.VMEM((_CH, d), jnp.float32),
          pltpu.VMEM_SHARED((n_pad, d), jnp.float32),
          pltpu.SemaphoreType.DMA,
      ],
  )
  def agg_kernel(src_hbm, dst_hbm, p_hbm, zeros_hbm, out_hbm, sidx_v,
                 didx_v, rows_v, acc_sh, sem):
    c = lax.axis_index("c")
    s = lax.axis_index("s")
    wid = c * _NS + s
    pltpu.sync_copy(zeros_hbm.at[pl.ds(s * rpt, rpt)],
                    acc_sh.at[pl.ds(s * rpt, rpt)])
    plsc.subcore_barrier()

    def step(i, carry):
      base = wid * e_per_w + i * _CH
      pltpu.sync_copy(src_hbm.at[pl.ds(base, _CH)], sidx_v)
      pltpu.sync_copy(dst_hbm.at[pl.ds(base, _CH)], didx_v)
      pltpu.async_copy(p_hbm.at[sidx_v], rows_v, sem).wait()
      pltpu.sync_copy(rows_v, acc_sh.at[didx_v], add=True)
      return carry

    lax.fori_loop(0, n_chunks, step, 0)
    plsc.subcore_barrier()
    pltpu.sync_copy(acc_sh.at[pl.ds(s * rpt, rpt)],
                    out_hbm.at[c, pl.ds(s * rpt, rpt)])

  return agg_kernel


def kernel(x, edge_index, W_lin, b_lin, W1, b1, W2, b2):
  n, f_in = x.shape
  h1 = W_lin.shape[1]
  h2 = W1.shape[1]
  cdim = W2.shape[1]
  e = edge_index.shape[1]

  n_pad = ((n + 256) // 256) * 256
  group = _NW * _CH
  e_pad = ((e + group - 1) // group) * group
  pad = e_pad - e

  src = edge_index[0].astype(jnp.int32)
  dst = edge_index[1].astype(jnp.int32)
  if pad:
    src = jnp.concatenate([src, jnp.zeros((pad,), jnp.int32)])
    dst = jnp.concatenate(
        [dst, n + (jnp.arange(pad, dtype=jnp.int32) % (n_pad - n))])

  ones_ch = jnp.ones((_CH,), jnp.float32)
  zeros_n = jnp.zeros((n_pad,), jnp.float32)
  zeros_n1 = jnp.zeros((n_pad, h2), jnp.float32)
  zeros_n2 = jnp.zeros((n_pad, cdim), jnp.float32)

  deg_parts = _make_deg(e_pad, n_pad)(dst, ones_ch, zeros_n)
  deg0 = deg_parts[0].reshape(n_pad, 1)
  deg1 = deg_parts[1].reshape(n_pad, 1)

  br = 400
  grid = n // br

  def tc1_body(x_ref, wl_ref, bl_ref, w1_ref, d0_ref, d1_ref, p1s_ref,
               dinv_ref):
    h = jnp.dot(x_ref[...], wl_ref[...], preferred_element_type=jnp.float32)
    h = jnp.maximum(h + bl_ref[...][None, :], 0.0)
    p = jnp.dot(h, w1_ref[...], preferred_element_type=jnp.float32)
    deg = d0_ref[...] + d1_ref[...] + 1.0
    dinv = lax.rsqrt(deg)
    p1s_ref[...] = p * dinv
    dinv_ref[...] = dinv

  p1s, dinv = pl.pallas_call(
      tc1_body,
      grid=(grid,),
      in_specs=[
          pl.BlockSpec((br, f_in), lambda i: (i, 0)),
          pl.BlockSpec((f_in, h1), lambda i: (0, 0)),
          pl.BlockSpec((h1,), lambda i: (0,)),
          pl.BlockSpec((h1, h2), lambda i: (0, 0)),
          pl.BlockSpec((br, 1), lambda i: (i, 0)),
          pl.BlockSpec((br, 1), lambda i: (i, 0)),
      ],
      out_specs=[
          pl.BlockSpec((br, h2), lambda i: (i, 0)),
          pl.BlockSpec((br, 1), lambda i: (i, 0)),
      ],
      out_shape=[
          jax.ShapeDtypeStruct((n, h2), jnp.float32),
          jax.ShapeDtypeStruct((n, 1), jnp.float32),
      ],
  )(x, W_lin, b_lin, W1, deg0, deg1)

  acc1 = _make_agg(e_pad, n_pad, h2)(src, dst, p1s, zeros_n1)

  def tc2_body(a_ref, p1s_ref, dinv_ref, b1_ref, w2_ref, p2s_ref):
    a = a_ref[...]
    dinv = dinv_ref[...]
    hagg = (a[0] + a[1] + p1s_ref[...]) * dinv + b1_ref[...][None, :]
    hagg = jnp.maximum(hagg, 0.0)
    p2s_ref[...] = jnp.dot(
        hagg, w2_ref[...], preferred_element_type=jnp.float32) * dinv

  p2s = pl.pallas_call(
      tc2_body,
      grid=(grid,),
      in_specs=[
          pl.BlockSpec((_NC, br, h2), lambda i: (0, i, 0)),
          pl.BlockSpec((br, h2), lambda i: (i, 0)),
          pl.BlockSpec((br, 1), lambda i: (i, 0)),
          pl.BlockSpec((h2,), lambda i: (0,)),
          pl.BlockSpec((h2, cdim), lambda i: (0, 0)),
      ],
      out_specs=pl.BlockSpec((br, cdim), lambda i: (i, 0)),
      out_shape=jax.ShapeDtypeStruct((n, cdim), jnp.float32),
  )(acc1, p1s, dinv, b1, W2)

  acc2 = _make_agg(e_pad, n_pad, cdim)(src, dst, p2s, zeros_n2)

  def tc3_body(a_ref, p2s_ref, dinv_ref, b2_ref, out_ref):
    a = a_ref[...]
    o = (a[0] + a[1] + p2s_ref[...]) * dinv_ref[...] + b2_ref[...][None, :]
    m = jnp.max(o, axis=1, keepdims=True)
    ex = jnp.exp(o - m)
    lse = jnp.log(jnp.sum(ex, axis=1, keepdims=True))
    out_ref[...] = o - m - lse

  out = pl.pallas_call(
      tc3_body,
      grid=(grid,),
      in_specs=[
          pl.BlockSpec((_NC, br, cdim), lambda i: (0, i, 0)),
          pl.BlockSpec((br, cdim), lambda i: (i, 0)),
          pl.BlockSpec((br, 1), lambda i: (i, 0)),
          pl.BlockSpec((cdim,), lambda i: (0,)),
      ],
      out_specs=pl.BlockSpec((br, cdim), lambda i: (i, 0)),
      out_shape=jax.ShapeDtypeStruct((n, cdim), jnp.float32),
  )(acc2, p2s, dinv, b2)

  return out


# speedup vs baseline: 10.6922x; 10.6922x over previous
import functools

import jax
import jax.numpy as jnp
from jax import lax
from jax.experimental import pallas as pl
from jax.experimental.pallas import tpu as pltpu
from jax.experimental.pallas import tpu_sc as plsc

_NC = 2
_NS = 16
_NW = _NC * _NS
_CH = 128


def _make_deg(e_pad, n_pad):
  e_per_w = e_pad // _NW
  n_chunks = e_per_w // _CH
  rpt = n_pad // _NS
  mesh = plsc.VectorSubcoreMesh(core_axis_name="c", subcore_axis_name="s")

  @functools.partial(
      pl.kernel,
      mesh=mesh,
      out_type=jax.ShapeDtypeStruct((_NC, n_pad), jnp.float32),
      scratch_types=[
          pltpu.VMEM((_CH,), jnp.int32),
          pltpu.VMEM((_CH,), jnp.float32),
          pltpu.VMEM_SHARED((n_pad,), jnp.float32),
          pltpu.SemaphoreType.DMA,
      ],
  )
  def deg_kernel(dst_hbm, ones_hbm, zeros_hbm, out_hbm, idx_v, ones_v,
                 acc_sh, sem):
    c = lax.axis_index("c")
    s = lax.axis_index("s")
    wid = c * _NS + s
    pltpu.sync_copy(zeros_hbm.at[pl.ds(s * rpt, rpt)],
                    acc_sh.at[pl.ds(s * rpt, rpt)])
    pltpu.sync_copy(ones_hbm, ones_v)
    plsc.subcore_barrier()

    def step(i, carry):
      base = wid * e_per_w + i * _CH
      pltpu.sync_copy(dst_hbm.at[pl.ds(base, _CH)], idx_v)
      pltpu.sync_copy(ones_v, acc_sh.at[idx_v], add=True)
      return carry

    lax.fori_loop(0, n_chunks, step, 0)
    plsc.subcore_barrier()
    pltpu.sync_copy(acc_sh.at[pl.ds(s * rpt, rpt)],
                    out_hbm.at[c, pl.ds(s * rpt, rpt)])

  return deg_kernel


def _make_agg(e_pad, n_pad, d):
  e_per_w = e_pad // _NW
  n_chunks = e_per_w // _CH
  rpt = n_pad // _NS
  mesh = plsc.VectorSubcoreMesh(core_axis_name="c", subcore_axis_name="s")

  @functools.partial(
      pl.kernel,
      mesh=mesh,
      out_type=jax.ShapeDtypeStruct((_NC, n_pad, d), jnp.float32),
      scratch_types=[
          pltpu.VMEM((_CH,), jnp.int32),
          pltpu.VMEM((_CH,), jnp.int32),
          pltpu.VMEM((_CH, d), jnp.float32),
          pltpu.VMEM_SHARED((n_pad, d), jnp.float32),
          pltpu.SemaphoreType.DMA,
      ],
  )
  def agg_kernel(src_hbm, dst_hbm, p_hbm, zeros_hbm, out_hbm, sidx_v,
                 didx_v, rows_v, acc_sh, sem):
    c = lax.axis_index("c")
    s = lax.axis_index("s")
    wid = c * _NS + s
    pltpu.sync_copy(zeros_hbm.at[pl.ds(s * rpt, rpt)],
                    acc_sh.at[pl.ds(s * rpt, rpt)])
    plsc.subcore_barrier()

    def step(i, carry):
      base = wid * e_per_w + i * _CH
      pltpu.sync_copy(src_hbm.at[pl.ds(base, _CH)], sidx_v)
      pltpu.sync_copy(dst_hbm.at[pl.ds(base, _CH)], didx_v)
      pltpu.async_copy(p_hbm.at[sidx_v], rows_v, sem).wait()
      pltpu.sync_copy(rows_v, acc_sh.at[didx_v], add=True)
      return carry

    lax.fori_loop(0, n_chunks, step, 0)
    plsc.subcore_barrier()
    pltpu.sync_copy(acc_sh.at[pl.ds(s * rpt, rpt)],
                    out_hbm.at[c, pl.ds(s * rpt, rpt)])

  return agg_kernel


def kernel(x, edge_index, W_lin, b_lin, W1, b1, W2, b2):
  n, f_in = x.shape
  h1 = W_lin.shape[1]
  h2 = W1.shape[1]
  cdim = W2.shape[1]
  e = edge_index.shape[1]

  n_pad = ((n + 256) // 256) * 256
  group = _NW * _CH
  e_pad = ((e + group - 1) // group) * group
  pad = e_pad - e

  src = edge_index[0].astype(jnp.int32)
  dst = edge_index[1].astype(jnp.int32)
  if pad:
    src = jnp.concatenate([src, jnp.zeros((pad,), jnp.int32)])
    dst = jnp.concatenate(
        [dst, n + (jnp.arange(pad, dtype=jnp.int32) % (n_pad - n))])

  dpad = 128
  W1p = jnp.pad(W1, ((0, 0), (0, dpad - h2)))
  b1p = jnp.pad(b1, (0, dpad - h2))
  W2p = jnp.pad(W2, ((0, dpad - h2), (0, dpad - cdim)))

  ones_ch = jnp.ones((_CH,), jnp.float32)
  zeros_n = jnp.zeros((n_pad,), jnp.float32)
  zeros_nd = jnp.zeros((n_pad, dpad), jnp.float32)

  deg_parts = _make_deg(e_pad, n_pad)(dst, ones_ch, zeros_n)
  deg0 = deg_parts[0].reshape(n_pad, 1)
  deg1 = deg_parts[1].reshape(n_pad, 1)

  br = 400
  grid = n // br

  def tc1_body(x_ref, wl_ref, bl_ref, w1_ref, d0_ref, d1_ref, p1s_ref,
               dinv_ref):
    h = jnp.dot(x_ref[...], wl_ref[...], preferred_element_type=jnp.float32)
    h = jnp.maximum(h + bl_ref[...][None, :], 0.0)
    p = jnp.dot(h, w1_ref[...], preferred_element_type=jnp.float32)
    deg = d0_ref[...] + d1_ref[...] + 1.0
    dinv = lax.rsqrt(deg)
    p1s_ref[...] = p * dinv
    dinv_ref[...] = dinv

  p1s, dinv = pl.pallas_call(
      tc1_body,
      grid=(grid,),
      in_specs=[
          pl.BlockSpec((br, f_in), lambda i: (i, 0)),
          pl.BlockSpec((f_in, h1), lambda i: (0, 0)),
          pl.BlockSpec((h1,), lambda i: (0,)),
          pl.BlockSpec((h1, dpad), lambda i: (0, 0)),
          pl.BlockSpec((br, 1), lambda i: (i, 0)),
          pl.BlockSpec((br, 1), lambda i: (i, 0)),
      ],
      out_specs=[
          pl.BlockSpec((br, dpad), lambda i: (i, 0)),
          pl.BlockSpec((br, 1), lambda i: (i, 0)),
      ],
      out_shape=[
          jax.ShapeDtypeStruct((n, dpad), jnp.float32),
          jax.ShapeDtypeStruct((n, 1), jnp.float32),
      ],
  )(x, W_lin, b_lin, W1p, deg0, deg1)

  acc1 = _make_agg(e_pad, n_pad, dpad)(src, dst, p1s, zeros_nd)

  def tc2_body(a_ref, p1s_ref, dinv_ref, b1_ref, w2_ref, p2s_ref):
    a = a_ref[...]
    dinv = dinv_ref[...]
    hagg = (a[0] + a[1] + p1s_ref[...]) * dinv + b1_ref[...][None, :]
    hagg = jnp.maximum(hagg, 0.0)
    p2s_ref[...] = jnp.dot(
        hagg, w2_ref[...], preferred_element_type=jnp.float32) * dinv

  p2s = pl.pallas_call(
      tc2_body,
      grid=(grid,),
      in_specs=[
          pl.BlockSpec((_NC, br, dpad), lambda i: (0, i, 0)),
          pl.BlockSpec((br, dpad), lambda i: (i, 0)),
          pl.BlockSpec((br, 1), lambda i: (i, 0)),
          pl.BlockSpec((dpad,), lambda i: (0,)),
          pl.BlockSpec((dpad, dpad), lambda i: (0, 0)),
      ],
      out_specs=pl.BlockSpec((br, dpad), lambda i: (i, 0)),
      out_shape=jax.ShapeDtypeStruct((n, dpad), jnp.float32),
  )(acc1, p1s, dinv, b1p, W2p)

  acc2 = _make_agg(e_pad, n_pad, dpad)(src, dst, p2s, zeros_nd)

  def tc3_body(a_ref, p2s_ref, dinv_ref, b2_ref, out_ref):
    a = a_ref[...]
    agg = (a[0] + a[1] + p2s_ref[...])[:, :cdim]
    o = agg * dinv_ref[...] + b2_ref[...][None, :]
    m = jnp.max(o, axis=1, keepdims=True)
    ex = jnp.exp(o - m)
    lse = jnp.log(jnp.sum(ex, axis=1, keepdims=True))
    out_ref[...] = o - m - lse

  out = pl.pallas_call(
      tc3_body,
      grid=(grid,),
      in_specs=[
          pl.BlockSpec((_NC, br, dpad), lambda i: (0, i, 0)),
          pl.BlockSpec((br, dpad), lambda i: (i, 0)),
          pl.BlockSpec((br, 1), lambda i: (i, 0)),
          pl.BlockSpec((cdim,), lambda i: (0,)),
      ],
      out_specs=pl.BlockSpec((br, cdim), lambda i: (i, 0)),
      out_shape=jax.ShapeDtypeStruct((n, cdim), jnp.float32),
  )(acc2, p2s, dinv, b2)

  return out
